# Initial kernel scaffold; baseline (speedup 1.0000x reference)
import jax
import jax.numpy as jnp
from jax.experimental import pallas as pl


def kernel(nfeats, coordinates, efeats, edge_index, node_graph_ids, We1, be1, We2, be2, Wx, bx, Wh1, bh1, Wh2, bh2, Wm, bm, Wd, bd, Wq, bq, Wo, bo):
    raise NotImplementedError("write your pallas kernel here")


# speedup vs baseline: 2.7049x; 2.7049x over previous
import functools

import jax
import jax.numpy as jnp
import numpy as np
from jax import lax
from jax.experimental import pallas as pl
from jax.experimental.pallas import tpu as pltpu
from jax.experimental.pallas import tpu_sc as plsc

N = 10000
E = 320000
F = 128
De = 16
H = 128
G = 100
GP = 104
NCONV = 20

BE = 512
BN = 1000
CH = 80


def _silu(x):
    return x * (1.0 / (1.0 + jnp.exp(-x)))


def _ab_body(h_ref, wa_ref, wb_ref, be1_ref, a_ref, b_ref):
    h = h_ref[...]
    a_ref[...] = jnp.dot(h, wa_ref[...], preferred_element_type=jnp.float32)
    b_ref[...] = jnp.dot(h, wb_ref[...], preferred_element_type=jnp.float32) + be1_ref[...]


@functools.cache
def _ab_call():
    full = lambda i: (0, 0)
    return pl.pallas_call(
        _ab_body,
        grid=(N // BN,),
        in_specs=[
            pl.BlockSpec((BN, F), lambda i: (i, 0)),
            pl.BlockSpec((F, H), full),
            pl.BlockSpec((F, H), full),
            pl.BlockSpec((1, H), full),
        ],
        out_specs=[
            pl.BlockSpec((BN, H), lambda i: (i, 0)),
            pl.BlockSpec((BN, H), lambda i: (i, 0)),
        ],
        out_shape=[
            jax.ShapeDtypeStruct((N, H), jnp.float32),
            jax.ShapeDtypeStruct((N, H), jnp.float32),
        ],
        name="ab_tables",
    )


def _edge_body(pre_ref, e_ref, relp_ref, we1e_ref, wd2_ref, we2_ref, be2_ref,
               wx_ref, bx_ref, upd_ref):
    relp = relp_ref[...]
    d2 = jnp.sum(relp * relp, axis=1, keepdims=True)
    u = (pre_ref[...] + d2 * wd2_ref[...]
         + jnp.dot(e_ref[...][:, :De], we1e_ref[...],
                   preferred_element_type=jnp.float32))
    m = _silu(u)
    en = _silu(jnp.dot(m, we2_ref[...], preferred_element_type=jnp.float32)
               + be2_ref[...])
    w = jnp.tanh(jnp.dot(en, wx_ref[...], preferred_element_type=jnp.float32)
                 + bx_ref[...])
    relw = relp[:, :3] * w
    one = jnp.ones((BE, 1), jnp.float32)
    pad = jnp.zeros((BE, 12), jnp.float32)
    upd_ref[...] = jnp.concatenate([en, relw, one, pad], axis=1)


@functools.cache
def _edge_call():
    full = lambda i: (0, 0)
    return pl.pallas_call(
        _edge_body,
        grid=(E // BE,),
        in_specs=[
            pl.BlockSpec((BE, H), lambda i: (i, 0)),
            pl.BlockSpec((BE, 32), lambda i: (i, 0)),
            pl.BlockSpec((BE, 16), lambda i: (i, 0)),
            pl.BlockSpec((De, H), full),
            pl.BlockSpec((1, H), full),
            pl.BlockSpec((H, De), full),
            pl.BlockSpec((1, De), full),
            pl.BlockSpec((De, 1), full),
            pl.BlockSpec((1, 1), full),
        ],
        out_specs=pl.BlockSpec((BE, 32), lambda i: (i, 0)),
        out_shape=jax.ShapeDtypeStruct((E, 32), jnp.float32),
        name="edge_mlp",
    )


def _node_body(h_ref, xp_ref, acc0_ref, acc1_ref, wh1h_ref, wh1a_ref,
               bh1_ref, wh2_ref, bh2_ref, wa_ref, wb_ref, be1_ref,
               h_out, xp_out, a_out, b_out):
    h = h_ref[...]
    accs = acc0_ref[...] + acc1_ref[...]
    deg = jnp.maximum(accs[:, 19:20], 1.0)
    agg = accs[:, 0:De] / deg
    x3 = xp_ref[...][:, :3] + accs[:, 16:19] / deg
    xp_out[...] = jnp.concatenate([x3, jnp.zeros((BN, 125), jnp.float32)], axis=1)
    t = _silu(jnp.dot(h, wh1h_ref[...], preferred_element_type=jnp.float32)
              + jnp.dot(agg, wh1a_ref[...], preferred_element_type=jnp.float32)
              + bh1_ref[...])
    hn = h + jnp.dot(t, wh2_ref[...], preferred_element_type=jnp.float32) + bh2_ref[...]
    h_out[...] = hn
    a_out[...] = jnp.dot(hn, wa_ref[...], preferred_element_type=jnp.float32)
    b_out[...] = jnp.dot(hn, wb_ref[...], preferred_element_type=jnp.float32) + be1_ref[...]


@functools.cache
def _node_call():
    full = lambda i: (0, 0)
    blk = lambda i: (i, 0)
    return pl.pallas_call(
        _node_body,
        grid=(N // BN,),
        in_specs=[
            pl.BlockSpec((BN, H), blk),
            pl.BlockSpec((BN, 128), blk),
            pl.BlockSpec((BN, 32), blk),
            pl.BlockSpec((BN, 32), lambda i: (i + N // BN, 0)),
            pl.BlockSpec((H, H), full),
            pl.BlockSpec((De, H), full),
            pl.BlockSpec((1, H), full),
            pl.BlockSpec((H, F), full),
            pl.BlockSpec((1, F), full),
            pl.BlockSpec((F, H), full),
            pl.BlockSpec((F, H), full),
            pl.BlockSpec((1, H), full),
        ],
        out_specs=[pl.BlockSpec((BN, H), blk), pl.BlockSpec((BN, 128), blk),
                   pl.BlockSpec((BN, H), blk), pl.BlockSpec((BN, H), blk)],
        out_shape=[
            jax.ShapeDtypeStruct((N, H), jnp.float32),
            jax.ShapeDtypeStruct((N, 128), jnp.float32),
            jax.ShapeDtypeStruct((N, H), jnp.float32),
            jax.ShapeDtypeStruct((N, H), jnp.float32),
        ],
        name="node_update",
    )


def _r1_body(hm_ref, nf_ref, gid_ref, wm_ref, bm_ref,
             pm_out, gs_out, gc_out):
    i = pl.program_id(0)
    pmb = jnp.dot(hm_ref[...], wm_ref[...], preferred_element_type=jnp.float32) + bm_ref[...]
    mask = nf_ref[...][:, 0:1] == 1.0
    pmb = jnp.where(mask, jnp.abs(pmb), pmb)
    pm_out[...] = pmb
    gi = jax.lax.broadcasted_iota(jnp.int32, (BN, GP), 1)
    oh = (gid_ref[...] == gi).astype(jnp.float32)
    dn = (((0,), (0,)), ((), ()))
    ps = jax.lax.dot_general(oh, pmb, dn, preferred_element_type=jnp.float32)
    pc = jax.lax.dot_general(oh, jnp.ones((BN, 1), jnp.float32), dn,
                             preferred_element_type=jnp.float32)

    @pl.when(i == 0)
    def _():
        gs_out[...] = ps
        gc_out[...] = pc

    @pl.when(i != 0)
    def _():
        gs_out[...] += ps
        gc_out[...] += pc


@functools.cache
def _r1_call():
    full = lambda i: (0, 0)
    blk = lambda i: (i, 0)
    return pl.pallas_call(
        _r1_body,
        grid=(N // BN,),
        in_specs=[
            pl.BlockSpec((BN, H), blk),
            pl.BlockSpec((BN, F), blk),
            pl.BlockSpec((BN, 1), blk),
            pl.BlockSpec((F, 1), full),
            pl.BlockSpec((1, 1), full),
        ],
        out_specs=[pl.BlockSpec((BN, 1), blk), pl.BlockSpec((GP, 1), full),
                   pl.BlockSpec((GP, 1), full)],
        out_shape=[
            jax.ShapeDtypeStruct((N, 1), jnp.float32),
            jax.ShapeDtypeStruct((GP, 1), jnp.float32),
            jax.ShapeDtypeStruct((GP, 1), jnp.float32),
        ],
        name="readout_pm",
    )


def _r2_body(pm_ref, gs_ref, gc_ref, gid_ref, hd_ref, hq_ref, ho_ref,
             wd_ref, bd_ref, wq_ref, bq_ref, wo_ref, bo_ref, out_ref):
    gs = gs_ref[...]
    fv = gs / jnp.maximum(gc_ref[...], 1.0)
    fv = jnp.where(jnp.abs(gs) < 0.01, 0.0, fv)
    gi = jax.lax.broadcasted_iota(jnp.int32, (BN, GP), 1)
    oh = (gid_ref[...] == gi).astype(jnp.float32)
    pm = pm_ref[...] - jnp.dot(oh, fv, preferred_element_type=jnp.float32)
    pd = jnp.dot(hd_ref[...], wd_ref[...], preferred_element_type=jnp.float32) + bd_ref[...]
    pq = jnp.dot(hq_ref[...], wq_ref[...], preferred_element_type=jnp.float32) + bq_ref[...]
    mt = (pq[:, 0:1] + pq[:, 3:4] + pq[:, 5:6]) / 3.0
    pq = jnp.concatenate([pq[:, 0:1] - mt, pq[:, 1:3], pq[:, 3:4] - mt,
                          pq[:, 4:5], pq[:, 5:6] - mt], axis=1)
    po = jnp.dot(ho_ref[...], wo_ref[...], preferred_element_type=jnp.float32) + bo_ref[...]
    m0 = (po[:, 0:1] + po[:, 3:4] + po[:, 5:6]) / 3.0
    m1 = (po[:, 6:7] + po[:, 1:2] + po[:, 8:9]) / 3.0
    m2 = (po[:, 9:10] + po[:, 2:3] + po[:, 7:8]) / 3.0
    po = jnp.concatenate([po[:, 0:1] - m0, po[:, 1:2] - m1, po[:, 2:3] - m2,
                          po[:, 3:4] - m0, po[:, 4:5], po[:, 5:6] - m0,
                          po[:, 6:7] - m1, po[:, 7:8] - m2, po[:, 8:9] - m1,
                          po[:, 9:10] - m2], axis=1)
    out_ref[...] = jnp.concatenate([pm, pd, pq, po], axis=1)


@functools.cache
def _r2_call():
    full = lambda i: (0, 0)
    blk = lambda i: (i, 0)
    return pl.pallas_call(
        _r2_body,
        grid=(N // BN,),
        in_specs=[
            pl.BlockSpec((BN, 1), blk),
            pl.BlockSpec((GP, 1), full),
            pl.BlockSpec((GP, 1), full),
            pl.BlockSpec((BN, 1), blk),
            pl.BlockSpec((BN, H), blk),
            pl.BlockSpec((BN, H), blk),
            pl.BlockSpec((BN, H), blk),
            pl.BlockSpec((F, 3), full),
            pl.BlockSpec((1, 3), full),
            pl.BlockSpec((F, 6), full),
            pl.BlockSpec((1, 6), full),
            pl.BlockSpec((F, 10), full),
            pl.BlockSpec((1, 10), full),
        ],
        out_specs=pl.BlockSpec((BN, 20), blk),
        out_shape=jax.ShapeDtypeStruct((N, 20), jnp.float32),
        name="readout_final",
    )


@functools.cache
def _sc_mesh():
    return plsc.VectorSubcoreMesh(core_axis_name="c", subcore_axis_name="s")


_NW = 32
_EW = E // _NW
_NCHUNK = _EW // CH


def _gather_body(a_hbm, b_hbm, xp_hbm, src_hbm, dst_hbm,
                 pre_hbm, relp_hbm,
                 srcv, dstv, arows, brows, xsr, xdr, relbuf, sem):
    cid = lax.axis_index("c")
    sid = lax.axis_index("s")
    wid = sid * 2 + cid
    base0 = wid * _EW

    def chunk(g, c):
        base = base0 + g * CH
        pltpu.sync_copy(src_hbm.at[pl.ds(base, CH)], srcv)
        pltpu.sync_copy(dst_hbm.at[pl.ds(base, CH)], dstv)
        pltpu.async_copy(a_hbm.at[srcv], arows, sem).wait()
        pltpu.async_copy(b_hbm.at[dstv], brows, sem).wait()
        pltpu.async_copy(xp_hbm.at[srcv], xsr, sem).wait()
        pltpu.async_copy(xp_hbm.at[dstv], xdr, sem).wait()

        def edge(i, cc):
            s16 = pl.ds(0, 16)
            relbuf[i, :] = xsr[i, s16] - xdr[i, s16]
            for j in range(8):
                sl = pl.ds(16 * j, 16)
                arows[i, sl] = arows[i, sl] + brows[i, sl]
            return cc

        lax.fori_loop(0, CH, edge, 0)
        pltpu.sync_copy(arows, pre_hbm.at[pl.ds(base, CH)])
        pltpu.sync_copy(relbuf, relp_hbm.at[pl.ds(base, CH)])
        return c

    lax.fori_loop(0, _NCHUNK, chunk, 0)


@functools.cache
def _gather_call():
    return pl.kernel(
        _gather_body,
        out_type=[
            jax.ShapeDtypeStruct((E, H), jnp.float32),
            jax.ShapeDtypeStruct((E, 16), jnp.float32),
        ],
        mesh=_sc_mesh(),
        scratch_types=[
            pltpu.VMEM((CH,), jnp.int32),
            pltpu.VMEM((CH,), jnp.int32),
            pltpu.VMEM((CH, H), jnp.float32),
            pltpu.VMEM((CH, H), jnp.float32),
            pltpu.VMEM((CH, 128), jnp.float32),
            pltpu.VMEM((CH, 128), jnp.float32),
            pltpu.VMEM((CH, 16), jnp.float32),
            pltpu.SemaphoreType.DMA,
        ],
        name="sc_edge_gather",
    )


_RPS = N // 16


def _scatter_body(upd_hbm, dst_hbm, acc_hbm, idxv, updv, zbuf, shared, sem):
    cid = lax.axis_index("c")
    sid = lax.axis_index("s")
    wid = sid * 2 + cid
    base0 = wid * _EW

    def zrow(i, c):
        zbuf[i, pl.ds(0, 16)] = jnp.zeros((16,), jnp.float32)
        zbuf[i, pl.ds(16, 16)] = jnp.zeros((16,), jnp.float32)
        return c

    lax.fori_loop(0, 200, zrow, 0)

    @pl.when(sid < 10)
    def _():
        for k in range(5):
            pltpu.sync_copy(zbuf, shared.at[pl.ds(sid * 1000 + k * 200, 200)])

    plsc.subcore_barrier()

    def chunk(g, c):
        base = base0 + g * CH
        pltpu.sync_copy(dst_hbm.at[pl.ds(base, CH)], idxv)
        pltpu.sync_copy(upd_hbm.at[pl.ds(base, CH)], updv)
        pltpu.sync_copy(updv, shared.at[idxv], add=True)
        return c

    lax.fori_loop(0, _NCHUNK, chunk, 0)
    plsc.subcore_barrier()

    @pl.when(sid < 10)
    def _():
        pltpu.sync_copy(shared.at[pl.ds(sid * 1000, 1000)],
                        acc_hbm.at[pl.ds(cid * N + sid * 1000, 1000)])


@functools.cache
def _scatter_call():
    return pl.kernel(
        _scatter_body,
        out_type=jax.ShapeDtypeStruct((2 * N, 32), jnp.float32),
        mesh=_sc_mesh(),
        scratch_types=[
            pltpu.VMEM((CH,), jnp.int32),
            pltpu.VMEM((CH, 32), jnp.float32),
            pltpu.VMEM((200, 32), jnp.float32),
            pltpu.VMEM_SHARED((N, 32), jnp.float32),
            pltpu.SemaphoreType.DMA,
        ],
        name="sc_edge_scatter",
    )


def kernel(nfeats, coordinates, efeats, edge_index, node_graph_ids,
           We1, be1, We2, be2, Wx, bx, Wh1, bh1, Wh2, bh2,
           Wm, bm, Wd, bd, Wq, bq, Wo, bo):
    src = edge_index[0]
    dst = edge_index[1]
    gid2 = node_graph_ids.reshape(N, 1)
    xp0 = jnp.pad(coordinates, ((0, 0), (0, 125)))
    e0 = jnp.pad(efeats, ((0, 0), (0, 16)))

    We1a = We1[:, :F]
    We1b = We1[:, F:2 * F]
    We1e = We1[:, 2 * F:2 * F + De]
    wd2 = We1[:, 2 * F + De].reshape(NCONV, 1, H)
    be1r = be1.reshape(NCONV, 1, H)
    be2r = be2.reshape(NCONV, 1, De)
    bxr = bx.reshape(NCONV, 1, 1)
    Wh1h = Wh1[:, :F]
    Wh1a = Wh1[:, F:]
    bh1r = bh1.reshape(NCONV, 1, H)
    bh2r = bh2.reshape(NCONV, 1, F)

    hs_out = []
    for b in range(4):
        h, xp, e = nfeats, xp0, e0
        i0 = b * 5
        A, B = _ab_call()(h, We1a[i0], We1b[i0], be1r[i0])
        for l in range(5):
            i = i0 + l
            pre, relp = _gather_call()(A, B, xp, src, dst)
            upd = _edge_call()(pre, e, relp, We1e[i], wd2[i], We2[i], be2r[i],
                               Wx[i], bxr[i])
            acc = _scatter_call()(upd, dst)
            j = (i + 1) % NCONV
            h, xp, A, B = _node_call()(h, xp, acc, acc, Wh1h[i], Wh1a[i],
                                       bh1r[i], Wh2[i], bh2r[i],
                                       We1a[j], We1b[j], be1r[j])
            e = upd
        hs_out.append(h)

    h_mon, h_dip, h_quad, h_oct = hs_out
    pm_raw, gs, gc = _r1_call()(h_mon, nfeats, gid2, Wm, bm.reshape(1, 1))
    out = _r2_call()(pm_raw, gs, gc, gid2, h_dip, h_quad, h_oct,
                     Wd, bd.reshape(1, 3), Wq, bq.reshape(1, 6),
                     Wo, bo.reshape(1, 10))
    return out


# speedup vs baseline: 3.5898x; 1.3272x over previous
import functools

import jax
import jax.numpy as jnp
import numpy as np
from jax import lax
from jax.experimental import pallas as pl
from jax.experimental.pallas import tpu as pltpu
from jax.experimental.pallas import tpu_sc as plsc

N = 10000
E = 320000
F = 128
De = 16
H = 128
G = 100
GP = 104
NCONV = 20

BE = 512
BN = 1000
CH = 80


def _silu(x):
    return x * (1.0 / (1.0 + jnp.exp(-x)))


def _ab_body(h_ref, wa_ref, wb_ref, be1_ref, a_ref, b_ref):
    h = h_ref[...]
    a_ref[...] = jnp.dot(h, wa_ref[...], preferred_element_type=jnp.float32)
    b_ref[...] = jnp.dot(h, wb_ref[...], preferred_element_type=jnp.float32) + be1_ref[...]


@functools.cache
def _ab_call():
    full = lambda i: (0, 0)
    return pl.pallas_call(
        _ab_body,
        grid=(N // BN,),
        in_specs=[
            pl.BlockSpec((BN, F), lambda i: (i, 0)),
            pl.BlockSpec((F, H), full),
            pl.BlockSpec((F, H), full),
            pl.BlockSpec((1, H), full),
        ],
        out_specs=[
            pl.BlockSpec((BN, H), lambda i: (i, 0)),
            pl.BlockSpec((BN, H), lambda i: (i, 0)),
        ],
        out_shape=[
            jax.ShapeDtypeStruct((N, H), jnp.float32),
            jax.ShapeDtypeStruct((N, H), jnp.float32),
        ],
        name="ab_tables",
    )


def _edge_body(pre_ref, e_ref, relp_ref, we1e_ref, wd2_ref, we2_ref, be2_ref,
               wx_ref, bx_ref, upd_ref):
    relp = relp_ref[...]
    d2 = jnp.sum(relp * relp, axis=1, keepdims=True)
    u = (pre_ref[...] + d2 * wd2_ref[...]
         + jnp.dot(e_ref[...][:, :De], we1e_ref[...],
                   preferred_element_type=jnp.float32))
    m = _silu(u)
    en = _silu(jnp.dot(m, we2_ref[...], preferred_element_type=jnp.float32)
               + be2_ref[...])
    w = jnp.tanh(jnp.dot(en, wx_ref[...], preferred_element_type=jnp.float32)
                 + bx_ref[...])
    relw = relp[:, :3] * w
    one = jnp.ones((BE, 1), jnp.float32)
    pad = jnp.zeros((BE, 12), jnp.float32)
    upd_ref[...] = jnp.concatenate([en, relw, one, pad], axis=1)


@functools.cache
def _edge_call():
    full = lambda i: (0, 0)
    return pl.pallas_call(
        _edge_body,
        grid=(E // BE,),
        in_specs=[
            pl.BlockSpec((BE, H), lambda i: (i, 0)),
            pl.BlockSpec((BE, 32), lambda i: (i, 0)),
            pl.BlockSpec((BE, 16), lambda i: (i, 0)),
            pl.BlockSpec((De, H), full),
            pl.BlockSpec((1, H), full),
            pl.BlockSpec((H, De), full),
            pl.BlockSpec((1, De), full),
            pl.BlockSpec((De, 1), full),
            pl.BlockSpec((1, 1), full),
        ],
        out_specs=pl.BlockSpec((BE, 32), lambda i: (i, 0)),
        out_shape=jax.ShapeDtypeStruct((E, 32), jnp.float32),
        name="edge_mlp",
    )


def _node_body(h_ref, xp_ref, acc0_ref, acc1_ref, wh1h_ref, wh1a_ref,
               bh1_ref, wh2_ref, bh2_ref, wa_ref, wb_ref, be1_ref,
               h_out, xp_out, a_out, b_out):
    h = h_ref[...]
    accs = acc0_ref[...] + acc1_ref[...]
    deg = jnp.maximum(accs[:, 19:20], 1.0)
    agg = accs[:, 0:De] / deg
    x3 = xp_ref[...][:, :3] + accs[:, 16:19] / deg
    xp_out[...] = jnp.concatenate([x3, jnp.zeros((BN, 125), jnp.float32)], axis=1)
    t = _silu(jnp.dot(h, wh1h_ref[...], preferred_element_type=jnp.float32)
              + jnp.dot(agg, wh1a_ref[...], preferred_element_type=jnp.float32)
              + bh1_ref[...])
    hn = h + jnp.dot(t, wh2_ref[...], preferred_element_type=jnp.float32) + bh2_ref[...]
    h_out[...] = hn
    a_out[...] = jnp.dot(hn, wa_ref[...], preferred_element_type=jnp.float32)
    b_out[...] = jnp.dot(hn, wb_ref[...], preferred_element_type=jnp.float32) + be1_ref[...]


@functools.cache
def _node_call():
    full = lambda i: (0, 0)
    blk = lambda i: (i, 0)
    return pl.pallas_call(
        _node_body,
        grid=(N // BN,),
        in_specs=[
            pl.BlockSpec((BN, H), blk),
            pl.BlockSpec((BN, 128), blk),
            pl.BlockSpec((BN, 32), blk),
            pl.BlockSpec((BN, 32), lambda i: (i + N // BN, 0)),
            pl.BlockSpec((H, H), full),
            pl.BlockSpec((De, H), full),
            pl.BlockSpec((1, H), full),
            pl.BlockSpec((H, F), full),
            pl.BlockSpec((1, F), full),
            pl.BlockSpec((F, H), full),
            pl.BlockSpec((F, H), full),
            pl.BlockSpec((1, H), full),
        ],
        out_specs=[pl.BlockSpec((BN, H), blk), pl.BlockSpec((BN, 128), blk),
                   pl.BlockSpec((BN, H), blk), pl.BlockSpec((BN, H), blk)],
        out_shape=[
            jax.ShapeDtypeStruct((N, H), jnp.float32),
            jax.ShapeDtypeStruct((N, 128), jnp.float32),
            jax.ShapeDtypeStruct((N, H), jnp.float32),
            jax.ShapeDtypeStruct((N, H), jnp.float32),
        ],
        name="node_update",
    )


def _r1_body(hm_ref, nf_ref, gid_ref, wm_ref, bm_ref,
             pm_out, gs_out, gc_out):
    i = pl.program_id(0)
    pmb = jnp.dot(hm_ref[...], wm_ref[...], preferred_element_type=jnp.float32) + bm_ref[...]
    mask = nf_ref[...][:, 0:1] == 1.0
    pmb = jnp.where(mask, jnp.abs(pmb), pmb)
    pm_out[...] = pmb
    gi = jax.lax.broadcasted_iota(jnp.int32, (BN, GP), 1)
    oh = (gid_ref[...] == gi).astype(jnp.float32)
    dn = (((0,), (0,)), ((), ()))
    ps = jax.lax.dot_general(oh, pmb, dn, preferred_element_type=jnp.float32)
    pc = jax.lax.dot_general(oh, jnp.ones((BN, 1), jnp.float32), dn,
                             preferred_element_type=jnp.float32)

    @pl.when(i == 0)
    def _():
        gs_out[...] = ps
        gc_out[...] = pc

    @pl.when(i != 0)
    def _():
        gs_out[...] += ps
        gc_out[...] += pc


@functools.cache
def _r1_call():
    full = lambda i: (0, 0)
    blk = lambda i: (i, 0)
    return pl.pallas_call(
        _r1_body,
        grid=(N // BN,),
        in_specs=[
            pl.BlockSpec((BN, H), blk),
            pl.BlockSpec((BN, F), blk),
            pl.BlockSpec((BN, 1), blk),
            pl.BlockSpec((F, 1), full),
            pl.BlockSpec((1, 1), full),
        ],
        out_specs=[pl.BlockSpec((BN, 1), blk), pl.BlockSpec((GP, 1), full),
                   pl.BlockSpec((GP, 1), full)],
        out_shape=[
            jax.ShapeDtypeStruct((N, 1), jnp.float32),
            jax.ShapeDtypeStruct((GP, 1), jnp.float32),
            jax.ShapeDtypeStruct((GP, 1), jnp.float32),
        ],
        name="readout_pm",
    )


def _r2_body(pm_ref, gs_ref, gc_ref, gid_ref, hd_ref, hq_ref, ho_ref,
             wd_ref, bd_ref, wq_ref, bq_ref, wo_ref, bo_ref, out_ref):
    gs = gs_ref[...]
    fv = gs / jnp.maximum(gc_ref[...], 1.0)
    fv = jnp.where(jnp.abs(gs) < 0.01, 0.0, fv)
    gi = jax.lax.broadcasted_iota(jnp.int32, (BN, GP), 1)
    oh = (gid_ref[...] == gi).astype(jnp.float32)
    pm = pm_ref[...] - jnp.dot(oh, fv, preferred_element_type=jnp.float32)
    pd = jnp.dot(hd_ref[...], wd_ref[...], preferred_element_type=jnp.float32) + bd_ref[...]
    pq = jnp.dot(hq_ref[...], wq_ref[...], preferred_element_type=jnp.float32) + bq_ref[...]
    mt = (pq[:, 0:1] + pq[:, 3:4] + pq[:, 5:6]) / 3.0
    pq = jnp.concatenate([pq[:, 0:1] - mt, pq[:, 1:3], pq[:, 3:4] - mt,
                          pq[:, 4:5], pq[:, 5:6] - mt], axis=1)
    po = jnp.dot(ho_ref[...], wo_ref[...], preferred_element_type=jnp.float32) + bo_ref[...]
    m0 = (po[:, 0:1] + po[:, 3:4] + po[:, 5:6]) / 3.0
    m1 = (po[:, 6:7] + po[:, 1:2] + po[:, 8:9]) / 3.0
    m2 = (po[:, 9:10] + po[:, 2:3] + po[:, 7:8]) / 3.0
    po = jnp.concatenate([po[:, 0:1] - m0, po[:, 1:2] - m1, po[:, 2:3] - m2,
                          po[:, 3:4] - m0, po[:, 4:5], po[:, 5:6] - m0,
                          po[:, 6:7] - m1, po[:, 7:8] - m2, po[:, 8:9] - m1,
                          po[:, 9:10] - m2], axis=1)
    out_ref[...] = jnp.concatenate([pm, pd, pq, po], axis=1)


@functools.cache
def _r2_call():
    full = lambda i: (0, 0)
    blk = lambda i: (i, 0)
    return pl.pallas_call(
        _r2_body,
        grid=(N // BN,),
        in_specs=[
            pl.BlockSpec((BN, 1), blk),
            pl.BlockSpec((GP, 1), full),
            pl.BlockSpec((GP, 1), full),
            pl.BlockSpec((BN, 1), blk),
            pl.BlockSpec((BN, H), blk),
            pl.BlockSpec((BN, H), blk),
            pl.BlockSpec((BN, H), blk),
            pl.BlockSpec((F, 3), full),
            pl.BlockSpec((1, 3), full),
            pl.BlockSpec((F, 6), full),
            pl.BlockSpec((1, 6), full),
            pl.BlockSpec((F, 10), full),
            pl.BlockSpec((1, 10), full),
        ],
        out_specs=pl.BlockSpec((BN, 20), blk),
        out_shape=jax.ShapeDtypeStruct((N, 20), jnp.float32),
        name="readout_final",
    )


@functools.cache
def _sc_mesh():
    return plsc.VectorSubcoreMesh(core_axis_name="c", subcore_axis_name="s")


_NW = 32
_EW = E // _NW
_NCHUNK = _EW // CH


def _gather_issue(g, b, a_hbm, b_hbm, xp_hbm, src_hbm, dst_hbm, base0,
                  srcv, dstv, arows, brows, xsr, xdr, sems):
    base = base0 + g * CH
    pltpu.sync_copy(src_hbm.at[pl.ds(base, CH)], srcv[b])
    pltpu.sync_copy(dst_hbm.at[pl.ds(base, CH)], dstv[b])
    pltpu.async_copy(a_hbm.at[srcv[b]], arows[b], sems[b])
    pltpu.async_copy(b_hbm.at[dstv[b]], brows[b], sems[b])
    pltpu.async_copy(xp_hbm.at[srcv[b]], xsr[b], sems[b])
    pltpu.async_copy(xp_hbm.at[dstv[b]], xdr[b], sems[b])


def _gather_drain_load(b, a_hbm, b_hbm, xp_hbm, srcv, dstv, arows, brows,
                       xsr, xdr, sems):
    pltpu.make_async_copy(a_hbm.at[srcv[b]], arows[b], sems[b]).wait()
    pltpu.make_async_copy(b_hbm.at[dstv[b]], brows[b], sems[b]).wait()
    pltpu.make_async_copy(xp_hbm.at[srcv[b]], xsr[b], sems[b]).wait()
    pltpu.make_async_copy(xp_hbm.at[dstv[b]], xdr[b], sems[b]).wait()


def _gather_process(b, arows, brows, xsr, xdr, relbuf):
    def edge(i, cc):
        s16 = pl.ds(0, 16)
        relbuf[b][i, :] = xsr[b][i, s16] - xdr[b][i, s16]
        for j in range(8):
            sl = pl.ds(16 * j, 16)
            arows[b][i, sl] = arows[b][i, sl] + brows[b][i, sl]
        return cc

    lax.fori_loop(0, CH, edge, 0)


def _gather_fire_out(g, b, base0, pre_hbm, relp_hbm, arows, relbuf, semo):
    base = base0 + g * CH
    pltpu.async_copy(arows[b], pre_hbm.at[pl.ds(base, CH)], semo[b])
    pltpu.async_copy(relbuf[b], relp_hbm.at[pl.ds(base, CH)], semo[b])


def _gather_drain_out(b, pre_hbm, relp_hbm, arows, relbuf, semo):
    pltpu.make_async_copy(arows[b], pre_hbm.at[pl.ds(0, CH)], semo[b]).wait()
    pltpu.make_async_copy(relbuf[b], relp_hbm.at[pl.ds(0, CH)], semo[b]).wait()


def _gather_body(a_hbm, b_hbm, xp_hbm, src_hbm, dst_hbm,
                 pre_hbm, relp_hbm,
                 srcv0, srcv1, dstv0, dstv1, arows0, arows1, brows0, brows1,
                 xsr0, xsr1, xdr0, xdr1, relbuf0, relbuf1,
                 sem0, sem1, semo0, semo1):
    cid = lax.axis_index("c")
    sid = lax.axis_index("s")
    wid = sid * 2 + cid
    base0 = wid * _EW
    srcv = (srcv0, srcv1)
    dstv = (dstv0, dstv1)
    arows = (arows0, arows1)
    brows = (brows0, brows1)
    xsr = (xsr0, xsr1)
    xdr = (xdr0, xdr1)
    relbuf = (relbuf0, relbuf1)
    sems = (sem0, sem1)
    semo = (semo0, semo1)

    ld = lambda g, b: _gather_issue(g, b, a_hbm, b_hbm, xp_hbm, src_hbm,
                                    dst_hbm, base0, srcv, dstv, arows, brows,
                                    xsr, xdr, sems)
    dl = lambda b: _gather_drain_load(b, a_hbm, b_hbm, xp_hbm, srcv, dstv,
                                      arows, brows, xsr, xdr, sems)
    pr = lambda b: _gather_process(b, arows, brows, xsr, xdr, relbuf)
    fo = lambda g, b: _gather_fire_out(g, b, base0, pre_hbm, relp_hbm, arows,
                                       relbuf, semo)
    do = lambda b: _gather_drain_out(b, pre_hbm, relp_hbm, arows, relbuf, semo)

    ld(0, 0)
    ld(1, 1)

    def body(gg, c):
        g = 2 * gg
        dl(0); pr(0); fo(g, 0)
        dl(1); pr(1); fo(g + 1, 1)
        do(0); ld(g + 2, 0)
        do(1); ld(g + 3, 1)
        return c

    lax.fori_loop(0, (_NCHUNK - 3) // 2, body, 0)
    t = _NCHUNK - 3
    dl(0); pr(0); fo(t, 0)
    dl(1); pr(1); fo(t + 1, 1)
    do(0); ld(t + 2, 0)
    dl(0); pr(0); fo(t + 2, 0)
    do(0)
    do(1)


@functools.cache
def _gather_call():
    return pl.kernel(
        _gather_body,
        out_type=[
            jax.ShapeDtypeStruct((E, H), jnp.float32),
            jax.ShapeDtypeStruct((E, 16), jnp.float32),
        ],
        mesh=_sc_mesh(),
        scratch_types=(
            [pltpu.VMEM((CH,), jnp.int32)] * 4
            + [pltpu.VMEM((CH, H), jnp.float32)] * 4
            + [pltpu.VMEM((CH, 128), jnp.float32)] * 4
            + [pltpu.VMEM((CH, 16), jnp.float32)] * 2
            + [pltpu.SemaphoreType.DMA] * 4
        ),
        name="sc_edge_gather",
    )


_RPS = N // 16


def _scatter_body(upd_hbm, dst_hbm, acc_hbm, idxv, updv, zbuf, shared, sem):
    cid = lax.axis_index("c")
    sid = lax.axis_index("s")
    wid = sid * 2 + cid
    base0 = wid * _EW

    def zrow(i, c):
        zbuf[i, pl.ds(0, 16)] = jnp.zeros((16,), jnp.float32)
        zbuf[i, pl.ds(16, 16)] = jnp.zeros((16,), jnp.float32)
        return c

    lax.fori_loop(0, 200, zrow, 0)

    @pl.when(sid < 10)
    def _():
        for k in range(5):
            pltpu.sync_copy(zbuf, shared.at[pl.ds(sid * 1000 + k * 200, 200)])

    plsc.subcore_barrier()

    def chunk(g, c):
        base = base0 + g * CH
        pltpu.sync_copy(dst_hbm.at[pl.ds(base, CH)], idxv)
        pltpu.sync_copy(upd_hbm.at[pl.ds(base, CH)], updv)
        pltpu.sync_copy(updv, shared.at[idxv], add=True)
        return c

    lax.fori_loop(0, _NCHUNK, chunk, 0)
    plsc.subcore_barrier()

    @pl.when(sid < 10)
    def _():
        pltpu.sync_copy(shared.at[pl.ds(sid * 1000, 1000)],
                        acc_hbm.at[pl.ds(cid * N + sid * 1000, 1000)])


@functools.cache
def _scatter_call():
    return pl.kernel(
        _scatter_body,
        out_type=jax.ShapeDtypeStruct((2 * N, 32), jnp.float32),
        mesh=_sc_mesh(),
        scratch_types=[
            pltpu.VMEM((CH,), jnp.int32),
            pltpu.VMEM((CH, 32), jnp.float32),
            pltpu.VMEM((200, 32), jnp.float32),
            pltpu.VMEM_SHARED((N, 32), jnp.float32),
            pltpu.SemaphoreType.DMA,
        ],
        name="sc_edge_scatter",
    )


def kernel(nfeats, coordinates, efeats, edge_index, node_graph_ids,
           We1, be1, We2, be2, Wx, bx, Wh1, bh1, Wh2, bh2,
           Wm, bm, Wd, bd, Wq, bq, Wo, bo):
    src = edge_index[0]
    dst = edge_index[1]
    gid2 = node_graph_ids.reshape(N, 1)
    xp0 = jnp.pad(coordinates, ((0, 0), (0, 125)))
    e0 = jnp.pad(efeats, ((0, 0), (0, 16)))

    We1a = We1[:, :F]
    We1b = We1[:, F:2 * F]
    We1e = We1[:, 2 * F:2 * F + De]
    wd2 = We1[:, 2 * F + De].reshape(NCONV, 1, H)
    be1r = be1.reshape(NCONV, 1, H)
    be2r = be2.reshape(NCONV, 1, De)
    bxr = bx.reshape(NCONV, 1, 1)
    Wh1h = Wh1[:, :F]
    Wh1a = Wh1[:, F:]
    bh1r = bh1.reshape(NCONV, 1, H)
    bh2r = bh2.reshape(NCONV, 1, F)

    hs_out = []
    for b in range(4):
        h, xp, e = nfeats, xp0, e0
        i0 = b * 5
        A, B = _ab_call()(h, We1a[i0], We1b[i0], be1r[i0])
        for l in range(5):
            i = i0 + l
            pre, relp = _gather_call()(A, B, xp, src, dst)
            upd = _edge_call()(pre, e, relp, We1e[i], wd2[i], We2[i], be2r[i],
                               Wx[i], bxr[i])
            acc = _scatter_call()(upd, dst)
            j = (i + 1) % NCONV
            h, xp, A, B = _node_call()(h, xp, acc, acc, Wh1h[i], Wh1a[i],
                                       bh1r[i], Wh2[i], bh2r[i],
                                       We1a[j], We1b[j], be1r[j])
            e = upd
        hs_out.append(h)

    h_mon, h_dip, h_quad, h_oct = hs_out
    pm_raw, gs, gc = _r1_call()(h_mon, nfeats, gid2, Wm, bm.reshape(1, 1))
    out = _r2_call()(pm_raw, gs, gc, gid2, h_dip, h_quad, h_oct,
                     Wd, bd.reshape(1, 3), Wq, bq.reshape(1, 6),
                     Wo, bo.reshape(1, 10))
    return out
